# Initial kernel scaffold; baseline (speedup 1.0000x reference)
import functools

import jax
import jax.numpy as jnp
from jax import lax
from jax.experimental import pallas as pl
from jax.experimental.pallas import tpu as pltpu
from jax.experimental.pallas import tpu_sc as plsc

N = 320000
D = 128
S = 10000
NC, NS = 2, 16
NW = NC * NS
RPW = N // NW
C = 80
NCHUNK = RPW // C
SEG_PER_TILE = S // NS
ZROWS = 125

RBLK = 8000


def _w_body(x_ref, w_ref, b_ref, o_ref):
    z = jnp.dot(x_ref[...], w_ref[...], preferred_element_type=jnp.float32)
    z = z + b_ref[...]
    o_ref[...] = 1.0 / (1.0 + jnp.exp(-z))


def _weights(x, W, b):
    return pl.pallas_call(
        _w_body,
        grid=(N // RBLK,),
        in_specs=[
            pl.BlockSpec((RBLK, D), lambda i: (i, 0)),
            pl.BlockSpec((D, 1), lambda i: (0, 0)),
            pl.BlockSpec((1, 1), lambda i: (0, 0)),
        ],
        out_specs=pl.BlockSpec((RBLK, 1), lambda i: (i, 0)),
        out_shape=jax.ShapeDtypeStruct((N, 1), jnp.float32),
    )(x, W, b.reshape(1, 1))


def _sc_body(x_hbm, w_hbm, ids_hbm, out_hbm, shared, xv, wv, iv, zv):
    cid = lax.axis_index("c")
    sid = lax.axis_index("s")
    wid = cid * NS + sid
    base = wid * RPW

    def zrow(i, _):
        zv[i // 8, pl.ds((i % 8) * 16, 16)] = jnp.zeros((16,), jnp.float32)
        return 0
    lax.fori_loop(0, ZROWS * 8, zrow, 0)
    for j in range(SEG_PER_TILE // ZROWS):
        pltpu.sync_copy(
            zv, shared.at[pl.ds(sid * SEG_PER_TILE + j * ZROWS, ZROWS)])
    plsc.subcore_barrier()

    def chunk(ci, _):
        rbase = base + ci * C
        pltpu.sync_copy(x_hbm.at[pl.ds(rbase, C)], xv)
        pltpu.sync_copy(w_hbm.at[pl.ds(rbase, C)], wv)
        pltpu.sync_copy(ids_hbm.at[pl.ds(rbase, C)], iv)

        def row(r, _):
            wr = wv[r]
            for k in range(D // 16):
                sl = pl.ds(k * 16, 16)
                xv[r, sl] = xv[r, sl] * wr
            return 0
        lax.fori_loop(0, C, row, 0)
        pltpu.sync_copy(xv, shared.at[iv], add=True)
        return 0
    lax.fori_loop(0, NCHUNK, chunk, 0)
    plsc.subcore_barrier()

    pltpu.sync_copy(
        shared.at[pl.ds(sid * SEG_PER_TILE, SEG_PER_TILE)],
        out_hbm.at[pl.ds(cid * S + sid * SEG_PER_TILE, SEG_PER_TILE)])


_sc_kernel = functools.partial(
    pl.kernel,
    out_type=jax.ShapeDtypeStruct((NC * S, D), jnp.float32),
    mesh=plsc.VectorSubcoreMesh(
        core_axis_name="c", subcore_axis_name="s",
        num_cores=NC, num_subcores=NS),
    scratch_types=[
        pltpu.VMEM_SHARED((S, D), jnp.float32),
        pltpu.VMEM((C, D), jnp.float32),
        pltpu.VMEM((C,), jnp.float32),
        pltpu.VMEM((C,), jnp.int32),
        pltpu.VMEM((ZROWS, D), jnp.float32),
    ],
)(_sc_body)


def kernel(x, segment_ids, W, b):
    ids = segment_ids.astype(jnp.int32)
    w = _weights(x, W, b).reshape(N)
    parts = _sc_kernel(x, w, ids)
    return parts[:S] + parts[S:]


# speedup vs baseline: 2.3401x; 2.3401x over previous
import functools

import jax
import jax.numpy as jnp
from jax import lax
from jax.experimental import pallas as pl
from jax.experimental.pallas import tpu as pltpu
from jax.experimental.pallas import tpu_sc as plsc

N = 320000
D = 128
S = 10000
NC, NS = 2, 16
NW = NC * NS
RPW = N // NW
C = 80
NCHUNK = RPW // C
SEG_STRIDE = 624
SEG_SPAN = 640
ZROWS = 80

RBLK = 8000


def _w_body(x_ref, w_ref, b_ref, o_ref):
    z = jnp.dot(x_ref[...], w_ref[...], preferred_element_type=jnp.float32)
    z = z + b_ref[...]
    o_ref[...] = 1.0 / (1.0 + jnp.exp(-z))


def _weights(x, W, b):
    return pl.pallas_call(
        _w_body,
        grid=(N // RBLK,),
        in_specs=[
            pl.BlockSpec((RBLK, D), lambda i: (i, 0)),
            pl.BlockSpec((D, 1), lambda i: (0, 0)),
            pl.BlockSpec((1, 1), lambda i: (0, 0)),
        ],
        out_specs=pl.BlockSpec((RBLK, 1), lambda i: (i, 0)),
        out_shape=jax.ShapeDtypeStruct((N, 1), jnp.float32),
    )(x, W, b.reshape(1, 1))


def _sc_body(x_hbm, w_hbm, ids_hbm, out_hbm, shared, xv, wv, iv, zv):
    cid = lax.axis_index("c")
    sid = lax.axis_index("s")
    wid = cid * NS + sid
    base = wid * RPW

    def zrow(i, _):
        zv[i // 8, pl.ds((i % 8) * 16, 16)] = jnp.zeros((16,), jnp.float32)
        return 0
    lax.fori_loop(0, ZROWS * 8, zrow, 0)
    for j in range(SEG_SPAN // ZROWS):
        pltpu.sync_copy(
            zv, shared.at[pl.ds(sid * SEG_STRIDE + j * ZROWS, ZROWS)])
    plsc.subcore_barrier()

    def chunk(ci, _):
        rbase = base + ci * C
        pltpu.sync_copy(x_hbm.at[pl.ds(rbase, C)], xv)
        pltpu.sync_copy(w_hbm.at[pl.ds(rbase, C)], wv)
        pltpu.sync_copy(ids_hbm.at[pl.ds(rbase, C)], iv)

        def grp(g, _):
            wvec = wv[pl.ds(g * 16, 16)]
            for j in range(16):
                wr = wvec[j]
                r = g * 16 + j
                for k in range(D // 16):
                    sl = pl.ds(k * 16, 16)
                    xv[r, sl] = xv[r, sl] * wr
            return 0
        lax.fori_loop(0, C // 16, grp, 0)
        pltpu.sync_copy(xv, shared.at[iv], add=True)
        return 0
    lax.fori_loop(0, NCHUNK, chunk, 0)
    plsc.subcore_barrier()

    pltpu.sync_copy(
        shared.at[pl.ds(sid * SEG_STRIDE, SEG_SPAN)],
        out_hbm.at[pl.ds(cid * S + sid * SEG_STRIDE, SEG_SPAN)])


_sc_kernel = functools.partial(
    pl.kernel,
    out_type=jax.ShapeDtypeStruct((NC * S, D), jnp.float32),
    mesh=plsc.VectorSubcoreMesh(
        core_axis_name="c", subcore_axis_name="s",
        num_cores=NC, num_subcores=NS),
    scratch_types=[
        pltpu.VMEM_SHARED((S, D), jnp.float32),
        pltpu.VMEM((C, D), jnp.float32),
        pltpu.VMEM((C,), jnp.float32),
        pltpu.VMEM((C,), jnp.int32),
        pltpu.VMEM((ZROWS, D), jnp.float32),
    ],
)(_sc_body)


def kernel(x, segment_ids, W, b):
    ids = segment_ids.astype(jnp.int32)
    w = _weights(x, W, b).reshape(N)
    parts = _sc_kernel(x, w, ids)
    return parts[:S] + parts[S:]


# speedup vs baseline: 4.0410x; 1.7268x over previous
import functools

import jax
import jax.numpy as jnp
from jax import lax
from jax.experimental import pallas as pl
from jax.experimental.pallas import tpu as pltpu
from jax.experimental.pallas import tpu_sc as plsc

N = 320000
D = 128
S = 10000
NC, NS = 2, 16
NW = NC * NS
RPW = N // NW
C = 80
NCHUNK = RPW // C
SEG_STRIDE = 624
SEG_SPAN = 640
ZROWS = 40

RBLK = 8000


def _w_body(x_ref, w_ref, b_ref, o_ref):
    z = jnp.dot(x_ref[...], w_ref[...], preferred_element_type=jnp.float32)
    z = z + b_ref[...]
    o_ref[...] = 1.0 / (1.0 + jnp.exp(-z))


def _weights(x, W, b):
    return pl.pallas_call(
        _w_body,
        grid=(N // RBLK,),
        in_specs=[
            pl.BlockSpec((RBLK, D), lambda i: (i, 0)),
            pl.BlockSpec((D, 1), lambda i: (0, 0)),
            pl.BlockSpec((1, 1), lambda i: (0, 0)),
        ],
        out_specs=pl.BlockSpec((RBLK, 1), lambda i: (i, 0)),
        out_shape=jax.ShapeDtypeStruct((N, 1), jnp.float32),
    )(x, W, b.reshape(1, 1))


NBUF = 4
NROUND = (NCHUNK - 1) // NBUF


def _sc_body(x_hbm, w_hbm, ids_hbm, out_hbm, shared, xv, wv, iv, zv,
             semx, semw, semi, semsc):
    cid = lax.axis_index("c")
    sid = lax.axis_index("s")
    wid = cid * NS + sid
    base = wid * RPW

    def issue_loads(ci, b):
        rbase = base + ci * C
        pltpu.async_copy(x_hbm.at[pl.ds(rbase, C)], xv.at[b], semx.at[b])
        pltpu.async_copy(w_hbm.at[pl.ds(rbase, C)], wv.at[b], semw.at[b])
        pltpu.async_copy(ids_hbm.at[pl.ds(rbase, C)], iv.at[b], semi.at[b])

    def wait_loads(ci, b):
        rbase = base + ci * C
        pltpu.make_async_copy(
            x_hbm.at[pl.ds(rbase, C)], xv.at[b], semx.at[b]).wait()
        pltpu.make_async_copy(
            w_hbm.at[pl.ds(rbase, C)], wv.at[b], semw.at[b]).wait()
        pltpu.make_async_copy(
            ids_hbm.at[pl.ds(rbase, C)], iv.at[b], semi.at[b]).wait()

    def wait_scatter(b):
        pltpu.make_async_copy(
            xv.at[b], shared.at[iv.at[b]], semsc.at[b]).wait()

    def zrow(i, _):
        zv[i // 8, pl.ds((i % 8) * 16, 16)] = jnp.zeros((16,), jnp.float32)
        return 0
    lax.fori_loop(0, ZROWS * 8, zrow, 0)
    for j in range(SEG_SPAN // ZROWS):
        pltpu.sync_copy(
            zv, shared.at[pl.ds(sid * SEG_STRIDE + j * ZROWS, ZROWS)])
    plsc.subcore_barrier()

    def scale_and_scatter(b):
        def grp(gg, _):
            wvec = wv[b, pl.ds(gg * 16, 16)]
            for j in range(16):
                wr = wvec[j]
                r = gg * 16 + j
                for k in range(D // 16):
                    sl = pl.ds(k * 16, 16)
                    xv[b, r, sl] = xv[b, r, sl] * wr
            return 0
        lax.fori_loop(0, C // 16, grp, 0)
        pltpu.sync_copy(xv.at[b], shared.at[iv.at[b]], add=True)

    for b in range(NBUF - 1):
        issue_loads(b, b)

    def piperound(g, _):
        for b in range(NBUF):
            ci = g * NBUF + b
            wait_loads(ci, b)
            bn = (b + NBUF - 1) % NBUF
            nci = ci + NBUF - 1

            @pl.when(nci < NCHUNK)
            def _():
                issue_loads(nci, bn)
            scale_and_scatter(b)
        return 0
    lax.fori_loop(0, NROUND, piperound, 0)
    last = NCHUNK - 1
    lastb = last % NBUF
    wait_loads(last, lastb)
    scale_and_scatter(lastb)
    plsc.subcore_barrier()

    pltpu.sync_copy(
        shared.at[pl.ds(sid * SEG_STRIDE, SEG_SPAN)],
        out_hbm.at[pl.ds(cid * S + sid * SEG_STRIDE, SEG_SPAN)])


_sc_kernel = functools.partial(
    pl.kernel,
    out_type=jax.ShapeDtypeStruct((NC * S, D), jnp.float32),
    mesh=plsc.VectorSubcoreMesh(
        core_axis_name="c", subcore_axis_name="s",
        num_cores=NC, num_subcores=NS),
    scratch_types=[
        pltpu.VMEM_SHARED((S, D), jnp.float32),
        pltpu.VMEM((NBUF, C, D), jnp.float32),
        pltpu.VMEM((NBUF, C), jnp.float32),
        pltpu.VMEM((NBUF, C), jnp.int32),
        pltpu.VMEM((ZROWS, D), jnp.float32),
        pltpu.SemaphoreType.DMA((NBUF,)),
        pltpu.SemaphoreType.DMA((NBUF,)),
        pltpu.SemaphoreType.DMA((NBUF,)),
        pltpu.SemaphoreType.DMA((NBUF,)),
    ],
)(_sc_body)


def kernel(x, segment_ids, W, b):
    ids = segment_ids.astype(jnp.int32)
    w = _weights(x, W, b).reshape(N)
    parts = _sc_kernel(x, w, ids)
    return parts[:S] + parts[S:]
